# baseline (speedup 1.0000x reference)
import jax
import jax.numpy as jnp
from jax import lax
from jax.experimental import pallas as pl
from jax.experimental.pallas import tpu as pltpu
from jax.experimental.pallas import tpu_sc as plsc

_B, _S, _D = 16384, 200, 64
_N = _B * _S

_INFO = plsc.get_sparse_core_info()
_NC, _NS = _INFO.num_cores, _INFO.num_subcores
_NW = _NC * _NS
_PER_W = _N // _NW
_T = 800
_CHUNKS = _PER_W // _T
_GROUPS = _T // 16


def _sc_body(idx_hbm, tab_hbm, out_hbm,
             tab_v, idx_v0, idx_v1, out_v0, out_v1, wsem0, wsem1):
    wid = lax.axis_index("s") * _NC + lax.axis_index("c")
    base = wid * _PER_W
    idx_v = (idx_v0, idx_v1)
    out_v = (out_v0, out_v1)
    wsem = (wsem0, wsem1)

    pltpu.sync_copy(tab_hbm, tab_v)
    lane = jnp.arange(16, dtype=jnp.int32)
    pos_col = lane * _D

    def chunk(g, _):
        for b in range(2):
            i = 2 * g + b
            tok0 = base + i * _T

            @pl.when(g >= 1)
            def _():
                pltpu.make_async_copy(
                    out_v[b], out_hbm.at[pl.ds(tok0 * _D, _T * _D)],
                    wsem[b]).wait()

            pltpu.sync_copy(idx_hbm.at[pl.ds(tok0, _T)], idx_v[b])

            def group(gi, pos0):
                off = pl.multiple_of(gi * 16, 16)
                idxg = idx_v[b][pl.ds(off, 16)]
                flat0 = idxg << 6
                for j in range(_D):
                    val = plsc.load_gather(tab_v, [flat0 + j])
                    plsc.store_scatter(out_v[b], [pos0 + j], val)
                return pos0 + (16 * _D)

            lax.fori_loop(0, _GROUPS, group, pos_col)
            pltpu.async_copy(out_v[b],
                             out_hbm.at[pl.ds(tok0 * _D, _T * _D)], wsem[b])
        return ()

    lax.fori_loop(0, _CHUNKS // 2, chunk, ())
    for b in range(2):
        pltpu.make_async_copy(out_v[b], out_hbm.at[pl.ds(base * _D, _T * _D)],
                              wsem[b]).wait()


def kernel(inputs, table):
    idx = inputs.astype(jnp.int32).reshape(_N)
    tab = table.reshape(7 * _D)
    out = pl.kernel(
        _sc_body,
        out_type=jax.ShapeDtypeStruct((_N * _D,), jnp.float32),
        mesh=plsc.VectorSubcoreMesh(core_axis_name="c", subcore_axis_name="s"),
        compiler_params=pltpu.CompilerParams(needs_layout_passes=False),
        scratch_types=[
            pltpu.VMEM((7 * _D,), jnp.float32),
            pltpu.VMEM((_T,), jnp.int32),
            pltpu.VMEM((_T,), jnp.int32),
            pltpu.VMEM((_T * _D,), jnp.float32),
            pltpu.VMEM((_T * _D,), jnp.float32),
            pltpu.SemaphoreType.DMA,
            pltpu.SemaphoreType.DMA,
        ],
    )(idx, tab)
    return out.reshape(_B, _S, _D)


# speedup vs baseline: 2.7071x; 2.7071x over previous
import jax
import jax.numpy as jnp
from jax import lax
from jax.experimental import pallas as pl
from jax.experimental.pallas import tpu as pltpu
from jax.experimental.pallas import tpu_sc as plsc

_B, _S, _D = 16384, 200, 64
_N = _B * _S

_INFO = plsc.get_sparse_core_info()
_NC, _NS = _INFO.num_cores, _INFO.num_subcores
_NW = _NC * _NS
_PER_W = _N // _NW
_T = 800
_CHUNKS = _PER_W // _T
_GROUPS = _T // 16


def _sc_body(idx_hbm, tab_hbm, out_hbm,
             tab_v, idx_v0, idx_v1, out_v0, out_v1, wsem0, wsem1):
    wid = lax.axis_index("s") * _NC + lax.axis_index("c")
    base = wid * _PER_W
    idx_v = (idx_v0, idx_v1)
    out_v = (out_v0, out_v1)
    wsem = (wsem0, wsem1)

    pltpu.sync_copy(tab_hbm, tab_v)
    iota = jnp.arange(16, dtype=jnp.int32)
    cols = [iota + 16 * k for k in range(_D // 16)]
    lanes = [jnp.full((16,), l, jnp.int32) for l in range(16)]
    splat_dnums = lax.GatherDimensionNumbers(
        offset_dims=(), collapsed_slice_dims=(0,), start_index_map=(0,))

    def chunk(g, _):
        for b in range(2):
            i = 2 * g + b
            tok0 = base + i * _T

            @pl.when(g >= 1)
            def _():
                pltpu.make_async_copy(
                    out_v[b], out_hbm.at[pl.ds(tok0 * _D, _T * _D)],
                    wsem[b]).wait()

            pltpu.sync_copy(idx_hbm.at[pl.ds(tok0, _T)], idx_v[b])

            def group(gi, _):
                off = pl.multiple_of(gi * 16, 16)
                obase = pl.multiple_of(gi * (16 * _D), 8)
                idxg = idx_v[b][pl.ds(off, 16)]
                for l in range(16):
                    splat = lax.gather(
                        idxg, lanes[l][:, None], splat_dnums, (1,),
                        mode=lax.GatherScatterMode.PROMISE_IN_BOUNDS)
                    flat0 = splat << 6
                    for k in range(_D // 16):
                        val = plsc.load_gather(tab_v, [flat0 + cols[k]])
                        out_v[b][pl.ds(obase + l * _D + 16 * k, 16)] = val
                return ()

            lax.fori_loop(0, _GROUPS, group, ())
            pltpu.async_copy(out_v[b],
                             out_hbm.at[pl.ds(tok0 * _D, _T * _D)], wsem[b])
        return ()

    lax.fori_loop(0, _CHUNKS // 2, chunk, ())
    for b in range(2):
        pltpu.make_async_copy(out_v[b], out_hbm.at[pl.ds(base * _D, _T * _D)],
                              wsem[b]).wait()


def kernel(inputs, table):
    idx = inputs.astype(jnp.int32).reshape(_N)
    tab = table.reshape(7 * _D)
    out = pl.kernel(
        _sc_body,
        out_type=jax.ShapeDtypeStruct((_N * _D,), jnp.float32),
        mesh=plsc.VectorSubcoreMesh(core_axis_name="c", subcore_axis_name="s"),
        compiler_params=pltpu.CompilerParams(needs_layout_passes=False),
        scratch_types=[
            pltpu.VMEM((7 * _D,), jnp.float32),
            pltpu.VMEM((_T,), jnp.int32),
            pltpu.VMEM((_T,), jnp.int32),
            pltpu.VMEM((_T * _D,), jnp.float32),
            pltpu.VMEM((_T * _D,), jnp.float32),
            pltpu.SemaphoreType.DMA,
            pltpu.SemaphoreType.DMA,
        ],
    )(idx, tab)
    return out.reshape(_B, _S, _D)
